# baseline (speedup 1.0000x reference)
import functools
import jax
import jax.numpy as jnp
from jax import lax
from jax.experimental import pallas as pl
from jax.experimental.pallas import tpu as pltpu
from jax.experimental.pallas import tpu_sc as plsc

_R = 160


def _make_sc_kernel(N, D):
    info = plsc.get_sparse_core_info()
    NC, NS = info.num_cores, info.num_subcores
    NW = NC * NS
    per_w = N // NW
    n_chunks = per_w // _R
    mesh = plsc.VectorSubcoreMesh(core_axis_name="c", subcore_axis_name="s")

    @functools.partial(
        pl.kernel, mesh=mesh,
        out_type=jax.ShapeDtypeStruct((N, D), jnp.float32),
        scratch_types=[
            pltpu.VMEM((per_w,), jnp.int32),
            pltpu.VMEM((_R, D), jnp.float32),
            pltpu.VMEM((_R, D), jnp.float32),
            pltpu.SemaphoreType.DMA,
            pltpu.SemaphoreType.DMA,
            pltpu.SemaphoreType.DMA,
        ],
    )
    def k(table_hbm, idx_hbm, out_hbm, idx_v, rows_a, rows_b, gsem, ssem_a,
          ssem_b):
        wid = lax.axis_index("s") * NC + lax.axis_index("c")
        base = wid * per_w
        pltpu.sync_copy(idx_hbm.at[pl.ds(base, per_w)], idx_v)
        bufs = (rows_a, rows_b)
        ssems = (ssem_a, ssem_b)

        def _scatter(i):
            return pltpu.make_async_copy(
                bufs[i % 2], out_hbm.at[pl.ds(base + i * _R, _R)], ssems[i % 2]
            )

        for i in range(n_chunks):
            if i >= 2:
                _scatter(i - 2).wait()
            pltpu.async_copy(
                table_hbm.at[idx_v.at[pl.ds(i * _R, _R)]], bufs[i % 2], gsem
            ).wait()
            _scatter(i).start()
        _scatter(n_chunks - 2).wait()
        _scatter(n_chunks - 1).wait()

    return k


def kernel(x, shift_indices):
    B, C, T, V = x.shape
    N = B * V * C
    xt = jnp.transpose(x, (0, 3, 1, 2)).reshape(N, T)
    si = shift_indices.astype(jnp.int32)
    ridx = (si.T[None, :, :] * C
            + jnp.arange(C, dtype=jnp.int32)[None, None, :]
            + (jnp.arange(B, dtype=jnp.int32) * (V * C))[:, None, None]
            ).reshape(N)
    out2 = _make_sc_kernel(N, T)(xt, ridx)
    return jnp.transpose(out2.reshape(B, V, C, T), (0, 2, 3, 1))
